# baseline (speedup 1.0000x reference)
import functools

import jax
import jax.numpy as jnp
from jax.experimental import pallas as pl
from jax.experimental.pallas import tpu as pltpu

N0 = 10000
E0 = 160000
HID = 128
NUM_CONVS = 3
POOL = [0.25, 0.25, 0.25]
K_NBR = 8
ALPHA = 0.2
BETA = 1.0
OUT_CH = 20

_BM = 512


def _linear_body(x_ref, w_ref, b_ref, r_ref, o_ref, *, act, residual, add_then_act):
    x = x_ref[...]
    w = w_ref[...]
    acc = jnp.dot(x.astype(jnp.bfloat16), w.astype(jnp.bfloat16),
                  preferred_element_type=jnp.float32)
    acc = acc + b_ref[...]
    if add_then_act and residual:
        acc = acc + r_ref[...]
    if act == "relu":
        acc = jnp.maximum(acc, 0.0)
    elif act == "leaky":
        acc = jnp.where(acc >= 0, acc, ALPHA * acc)
    if residual and not add_then_act:
        acc = acc + r_ref[...]
    o_ref[...] = acc


def _linear(x, w, b, act=None, residual=False, res_in=None, add_then_act=False):
    m, k = x.shape
    n = w.shape[1]
    mp = ((m + _BM - 1) // _BM) * _BM
    r = x if res_in is None else res_in
    if mp != m:
        x = jnp.pad(x, ((0, mp - m), (0, 0)))
        if residual:
            r = jnp.pad(r, ((0, mp - m), (0, 0)))
    grid = (mp // _BM,)
    in_specs = [
        pl.BlockSpec((_BM, k), lambda i: (i, 0)),
        pl.BlockSpec((k, n), lambda i: (0, 0)),
        pl.BlockSpec((1, n), lambda i: (0, 0)),
        pl.BlockSpec((_BM, n), lambda i: (i, 0)),
    ]
    out = pl.pallas_call(
        functools.partial(_linear_body, act=act, residual=residual,
                          add_then_act=add_then_act),
        grid=grid,
        in_specs=in_specs,
        out_specs=pl.BlockSpec((_BM, n), lambda i: (i, 0)),
        out_shape=jax.ShapeDtypeStruct((mp, n), jnp.float32),
    )(x, w, b.reshape(1, n), r if residual else jnp.zeros((mp, n), jnp.float32))
    return out[:m] if mp != m else out


def _leaky(x):
    return jnp.where(x >= 0, x, ALPHA * x)


def _relu(x):
    return jnp.maximum(x, 0.0)


_BQ = 256


def _knn_body(q_ref, rt_ref, o_ref, *, k, r_n):
    q = q_ref[...]
    rt = rt_ref[...]
    qq = jnp.sum(q * q, axis=1, keepdims=True)
    rsq = jnp.sum(rt * rt, axis=0, keepdims=True)
    qr = jnp.dot(q.astype(jnp.bfloat16), rt.astype(jnp.bfloat16),
                 preferred_element_type=jnp.float32)
    d = qq - 2.0 * qr + rsq
    iota = jax.lax.broadcasted_iota(jnp.int32, d.shape, 1)
    big = jnp.float32(jnp.inf)
    for j in range(k):
        dmin = jnp.min(d, axis=1, keepdims=True)
        sel = jnp.where(d == dmin, iota, r_n)
        ij = jnp.min(sel, axis=1)
        o_ref[:, j] = ij
        d = jnp.where(iota == ij[:, None], big, d)


def _knn_idx(q, r, k):
    qn, rn = q.shape[0], r.shape[0]
    qp = ((qn + _BQ - 1) // _BQ) * _BQ
    if qp != qn:
        q = jnp.pad(q, ((0, qp - qn), (0, 0)))
    rt = r.T
    idx = pl.pallas_call(
        functools.partial(_knn_body, k=k, r_n=rn),
        grid=(qp // _BQ,),
        in_specs=[
            pl.BlockSpec((_BQ, 3), lambda i: (i, 0)),
            pl.BlockSpec((3, rn), lambda i: (0, 0)),
        ],
        out_specs=pl.BlockSpec((_BQ, k), lambda i: (i, 0)),
        out_shape=jax.ShapeDtypeStruct((qp, k), jnp.int32),
    )(q, rt)
    return idx[:qn] if qp != qn else idx


def _gapl_edges(pos_s, x_s, ei, pos_d, x_d, p, name):
    dst, src = ei[0], ei[1]
    c = x_s.shape[1]
    w = p[name + "_W"]
    xw = _linear(x_s, w[:c], jnp.zeros((w.shape[1],), jnp.float32))
    m = _linear(pos_s[src] - pos_d[dst], w[c:], p[name + "_b"], act="leaky",
                residual=True, res_in=xw[src], add_then_act=True)
    n_d = x_d.shape[0]
    agg = jax.ops.segment_sum(m, dst, num_segments=n_d)
    deg = jax.ops.segment_sum(jnp.ones((dst.shape[0],), jnp.float32), dst, num_segments=n_d)
    agg = agg / jnp.clip(deg, 1.0)[:, None]
    out = BETA * x_d + agg
    for blk in ("_r0", "_r1"):
        out = _linear(out, p[name + blk + "_W"], p[name + blk + "_b"],
                      act="relu", residual=True)
    return out


def _gapl_knn(pos_s, x_s, idx, pos_d, x_d, p, name, mask=None):
    nk, k = idx.shape
    c = x_s.shape[1]
    w = p[name + "_W"]
    xw = _linear(x_s, w[:c], jnp.zeros((w.shape[1],), jnp.float32))
    pd = (pos_s[idx] - pos_d[:, None, :]).reshape(nk * k, 3)
    m = _linear(pd, w[c:], p[name + "_b"], act="leaky", residual=True,
                res_in=xw[idx].reshape(nk * k, -1),
                add_then_act=True).reshape(nk, k, -1)
    if mask is None:
        agg = m.sum(axis=1) / float(k)
    else:
        m = m * mask[:, :, None]
        deg = mask.sum(axis=1)
        agg = m.sum(axis=1) / jnp.clip(deg, 1.0)[:, None]
    out = BETA * x_d + agg
    for blk in ("_r0", "_r1"):
        out = _linear(out, p[name + blk + "_W"], p[name + blk + "_b"],
                      act="relu", residual=True)
    return out


def _mesh_enc(tok, p, pre):
    n = tok.shape[0]
    h = _linear(tok.reshape(n * 4, 3), p[pre + "_tok_W"], p[pre + "_tok_b"], act="relu")
    h = h.reshape(n, 4, -1) + p[pre + "_pos"][None]
    h = h.mean(axis=1)
    return _linear(h, p[pre + "_out_W"], p[pre + "_out_b"], act="relu")


def _knn_interpolate(x, pos_c, pos_f, k=3):
    idx = _knn_idx(pos_f, pos_c, k)
    d2 = jnp.sum((pos_f[:, None, :] - pos_c[idx]) ** 2, -1)
    w = 1.0 / (d2 + 1e-8)
    return jnp.sum(w[..., None] * x[idx], axis=1) / jnp.sum(w, axis=1, keepdims=True)


def kernel(pos, rgb, normals, params, edge_index, batch, ptr):
    p = params
    x_rgb = _mesh_enc(rgb.reshape(-1, 4, 3), p, "ce")
    x_nrm = _mesh_enc(normals.reshape(-1, 4, 3), p, "ne")
    x = jnp.concatenate([x_rgb, x_nrm, pos], -1)
    x = _linear(x, p["emb_W"], p["emb_b"], act="relu")
    x = _gapl_edges(pos, x, edge_index, pos, x, p, "g0")
    for blk in ("rm0", "rm1"):
        x = _linear(x, p[blk + "_W"], p[blk + "_b"], act="relu", residual=True)
    pos_down, x_down = [pos], [x]
    n = pos.shape[0]
    for i in range(NUM_CONVS):
        n_keep = int(n * POOL[i])
        perm = jax.random.permutation(jax.random.key(100 + i), n)[:n_keep]
        pos_pool = pos[perm]
        idx_h = _knn_idx(pos_pool, pos, K_NBR)
        x_hier = _gapl_knn(pos, x, idx_h, pos_pool, x[perm], p, f"hi{i}")
        idx_l = _knn_idx(pos_pool, pos_pool, K_NBR)
        sim = jnp.sum(x_hier[:, None, :] * x_hier[idx_l], -1)
        n_e = n_keep * K_NBR
        _, keep = jax.lax.top_k(sim.reshape(-1), n_e // 2)
        mask = jnp.zeros((n_e,), jnp.float32).at[keep].set(1.0).reshape(n_keep, K_NBR)
        x_local = _gapl_knn(pos_pool, x_hier, idx_l, pos_pool, x_hier, p, f"lo{i}",
                            mask=mask)
        x = jnp.concatenate([x_local, x_hier], -1)
        for blk in (f"rc{i}_0", f"rc{i}_1"):
            x = _linear(x, p[blk + "_W"], p[blk + "_b"], act="relu", residual=True)
        pos, n = pos_pool, n_keep
        pos_down.append(pos)
        x_down.append(x)
    xs = x_down[::-1]
    poss = pos_down[::-1]
    x_i = xs[0]
    for i in range(NUM_CONVS):
        up = _knn_interpolate(x_i, poss[i], poss[i + 1], 3)
        x_i = jnp.concatenate([xs[i + 1], up], -1)
        x_i = _linear(x_i, p[f"up{i}_W"], p[f"up{i}_b"], act="relu")
        x_i = _linear(x_i, p[f"up{i}_r_W"], p[f"up{i}_r_b"], act="relu", residual=True)
    y = _linear(x_i, p["m0_W"], p["m0_b"], act="relu")
    y = _linear(y, p["m1_W"], p["m1_b"], act="relu")
    return _linear(y, p["m2_W"], p["m2_b"])


# speedup vs baseline: 1.0709x; 1.0709x over previous
import functools

import jax
import jax.numpy as jnp
from jax.experimental import pallas as pl
from jax.experimental.pallas import tpu as pltpu

N0 = 10000
E0 = 160000
HID = 128
NUM_CONVS = 3
POOL = [0.25, 0.25, 0.25]
K_NBR = 8
ALPHA = 0.2
BETA = 1.0
OUT_CH = 20

_BM = 1024


def _linear_body(x_ref, w_ref, b_ref, r_ref, o_ref, *, act, residual, add_then_act):
    x = x_ref[...]
    w = w_ref[...]
    acc = jnp.dot(x.astype(jnp.bfloat16), w.astype(jnp.bfloat16),
                  preferred_element_type=jnp.float32)
    acc = acc + b_ref[...]
    if add_then_act and residual:
        acc = acc + r_ref[...]
    if act == "relu":
        acc = jnp.maximum(acc, 0.0)
    elif act == "leaky":
        acc = jnp.where(acc >= 0, acc, ALPHA * acc)
    if residual and not add_then_act:
        acc = acc + r_ref[...]
    o_ref[...] = acc


def _linear(x, w, b, act=None, residual=False, res_in=None, add_then_act=False):
    m, k = x.shape
    n = w.shape[1]
    mp = ((m + _BM - 1) // _BM) * _BM
    r = x if res_in is None else res_in
    if mp != m:
        x = jnp.pad(x, ((0, mp - m), (0, 0)))
        if residual:
            r = jnp.pad(r, ((0, mp - m), (0, 0)))
    grid = (mp // _BM,)
    in_specs = [
        pl.BlockSpec((_BM, k), lambda i: (i, 0)),
        pl.BlockSpec((k, n), lambda i: (0, 0)),
        pl.BlockSpec((1, n), lambda i: (0, 0)),
        pl.BlockSpec((_BM, n), lambda i: (i, 0)),
    ]
    out = pl.pallas_call(
        functools.partial(_linear_body, act=act, residual=residual,
                          add_then_act=add_then_act),
        grid=grid,
        in_specs=in_specs,
        out_specs=pl.BlockSpec((_BM, n), lambda i: (i, 0)),
        out_shape=jax.ShapeDtypeStruct((mp, n), jnp.float32),
    )(x, w, b.reshape(1, n), r if residual else jnp.zeros((mp, n), jnp.float32))
    return out[:m] if mp != m else out


def _leaky(x):
    return jnp.where(x >= 0, x, ALPHA * x)


def _relu(x):
    return jnp.maximum(x, 0.0)


_BQ = 256


def _knn_body(q_ref, rt_ref, o_ref, *, k, r_n):
    q = q_ref[...]
    rt = rt_ref[...]
    qq = jnp.sum(q * q, axis=1, keepdims=True)
    rsq = jnp.sum(rt * rt, axis=0, keepdims=True)
    qr = jnp.dot(q.astype(jnp.bfloat16), rt.astype(jnp.bfloat16),
                 preferred_element_type=jnp.float32)
    d = qq - 2.0 * qr + rsq
    iota = jax.lax.broadcasted_iota(jnp.int32, d.shape, 1)
    big = jnp.float32(jnp.inf)
    for j in range(k):
        dmin = jnp.min(d, axis=1, keepdims=True)
        sel = jnp.where(d == dmin, iota, r_n)
        ij = jnp.min(sel, axis=1)
        o_ref[:, j] = ij
        d = jnp.where(iota == ij[:, None], big, d)


def _knn_idx(q, r, k):
    qn, rn = q.shape[0], r.shape[0]
    qp = ((qn + _BQ - 1) // _BQ) * _BQ
    if qp != qn:
        q = jnp.pad(q, ((0, qp - qn), (0, 0)))
    rt = r.T
    idx = pl.pallas_call(
        functools.partial(_knn_body, k=k, r_n=rn),
        grid=(qp // _BQ,),
        in_specs=[
            pl.BlockSpec((_BQ, 3), lambda i: (i, 0)),
            pl.BlockSpec((3, rn), lambda i: (0, 0)),
        ],
        out_specs=pl.BlockSpec((_BQ, k), lambda i: (i, 0)),
        out_shape=jax.ShapeDtypeStruct((qp, k), jnp.int32),
    )(q, rt)
    return idx[:qn] if qp != qn else idx


def _gapl_edges(pos_s, x_s, ei, pos_d, x_d, p, name):
    dst, src = ei[0], ei[1]
    c = x_s.shape[1]
    w = p[name + "_W"]
    xw = _linear(x_s, w[:c], jnp.zeros((w.shape[1],), jnp.float32))
    pw = _linear(pos_s[src] - pos_d[dst], w[c:], p[name + "_b"])
    m = _leaky(xw[src] + pw)
    n_d = x_d.shape[0]
    agg = jax.ops.segment_sum(m, dst, num_segments=n_d)
    deg = jax.ops.segment_sum(jnp.ones((dst.shape[0],), jnp.float32), dst, num_segments=n_d)
    agg = agg / jnp.clip(deg, 1.0)[:, None]
    out = BETA * x_d + agg
    for blk in ("_r0", "_r1"):
        out = _linear(out, p[name + blk + "_W"], p[name + blk + "_b"],
                      act="relu", residual=True)
    return out


def _gapl_knn(pos_s, x_s, idx, pos_d, x_d, p, name, mask=None):
    nk, k = idx.shape
    c = x_s.shape[1]
    w = p[name + "_W"]
    xw = _linear(x_s, w[:c], jnp.zeros((w.shape[1],), jnp.float32))
    pd = (pos_s[idx] - pos_d[:, None, :]).reshape(nk * k, 3)
    pw = _linear(pd, w[c:], p[name + "_b"])
    m = _leaky(xw[idx].reshape(nk * k, -1) + pw).reshape(nk, k, -1)
    if mask is None:
        agg = m.sum(axis=1) / float(k)
    else:
        m = m * mask[:, :, None]
        deg = mask.sum(axis=1)
        agg = m.sum(axis=1) / jnp.clip(deg, 1.0)[:, None]
    out = BETA * x_d + agg
    for blk in ("_r0", "_r1"):
        out = _linear(out, p[name + blk + "_W"], p[name + blk + "_b"],
                      act="relu", residual=True)
    return out


def _mesh_enc(tok, p, pre):
    n = tok.shape[0]
    h = _linear(tok.reshape(n * 4, 3), p[pre + "_tok_W"], p[pre + "_tok_b"], act="relu")
    h = h.reshape(n, 4, -1) + p[pre + "_pos"][None]
    h = h.mean(axis=1)
    return _linear(h, p[pre + "_out_W"], p[pre + "_out_b"], act="relu")


def _knn_interpolate(x, pos_c, pos_f, k=3):
    idx = _knn_idx(pos_f, pos_c, k)
    d2 = jnp.sum((pos_f[:, None, :] - pos_c[idx]) ** 2, -1)
    w = 1.0 / (d2 + 1e-8)
    return jnp.sum(w[..., None] * x[idx], axis=1) / jnp.sum(w, axis=1, keepdims=True)


def kernel(pos, rgb, normals, params, edge_index, batch, ptr):
    p = params
    x_rgb = _mesh_enc(rgb.reshape(-1, 4, 3), p, "ce")
    x_nrm = _mesh_enc(normals.reshape(-1, 4, 3), p, "ne")
    x = jnp.concatenate([x_rgb, x_nrm, pos], -1)
    x = _linear(x, p["emb_W"], p["emb_b"], act="relu")
    x = _gapl_edges(pos, x, edge_index, pos, x, p, "g0")
    for blk in ("rm0", "rm1"):
        x = _linear(x, p[blk + "_W"], p[blk + "_b"], act="relu", residual=True)
    pos_down, x_down = [pos], [x]
    n = pos.shape[0]
    for i in range(NUM_CONVS):
        n_keep = int(n * POOL[i])
        perm = jax.random.permutation(jax.random.key(100 + i), n)[:n_keep]
        pos_pool = pos[perm]
        idx_h = _knn_idx(pos_pool, pos, K_NBR)
        x_hier = _gapl_knn(pos, x, idx_h, pos_pool, x[perm], p, f"hi{i}")
        idx_l = _knn_idx(pos_pool, pos_pool, K_NBR)
        sim = jnp.sum(x_hier[:, None, :] * x_hier[idx_l], -1)
        n_e = n_keep * K_NBR
        _, keep = jax.lax.top_k(sim.reshape(-1), n_e // 2)
        mask = jnp.zeros((n_e,), jnp.float32).at[keep].set(1.0).reshape(n_keep, K_NBR)
        x_local = _gapl_knn(pos_pool, x_hier, idx_l, pos_pool, x_hier, p, f"lo{i}",
                            mask=mask)
        x = jnp.concatenate([x_local, x_hier], -1)
        for blk in (f"rc{i}_0", f"rc{i}_1"):
            x = _linear(x, p[blk + "_W"], p[blk + "_b"], act="relu", residual=True)
        pos, n = pos_pool, n_keep
        pos_down.append(pos)
        x_down.append(x)
    xs = x_down[::-1]
    poss = pos_down[::-1]
    x_i = xs[0]
    for i in range(NUM_CONVS):
        up = _knn_interpolate(x_i, poss[i], poss[i + 1], 3)
        x_i = jnp.concatenate([xs[i + 1], up], -1)
        x_i = _linear(x_i, p[f"up{i}_W"], p[f"up{i}_b"], act="relu")
        x_i = _linear(x_i, p[f"up{i}_r_W"], p[f"up{i}_r_b"], act="relu", residual=True)
    y = _linear(x_i, p["m0_W"], p["m0_b"], act="relu")
    y = _linear(y, p["m1_W"], p["m1_b"], act="relu")
    return _linear(y, p["m2_W"], p["m2_b"])


# speedup vs baseline: 1.0823x; 1.0106x over previous
import functools

import jax
import jax.numpy as jnp
from jax.experimental import pallas as pl
from jax.experimental.pallas import tpu as pltpu

N0 = 10000
E0 = 160000
HID = 128
NUM_CONVS = 3
POOL = [0.25, 0.25, 0.25]
K_NBR = 8
ALPHA = 0.2
BETA = 1.0
OUT_CH = 20

_BM = 2048


def _linear_body(x_ref, w_ref, b_ref, r_ref, o_ref, *, act, residual, add_then_act):
    x = x_ref[...]
    w = w_ref[...]
    acc = jnp.dot(x.astype(jnp.bfloat16), w.astype(jnp.bfloat16),
                  preferred_element_type=jnp.float32)
    acc = acc + b_ref[...]
    if add_then_act and residual:
        acc = acc + r_ref[...]
    if act == "relu":
        acc = jnp.maximum(acc, 0.0)
    elif act == "leaky":
        acc = jnp.where(acc >= 0, acc, ALPHA * acc)
    if residual and not add_then_act:
        acc = acc + r_ref[...]
    o_ref[...] = acc


def _linear(x, w, b, act=None, residual=False, res_in=None, add_then_act=False):
    m, k = x.shape
    n = w.shape[1]
    mp = ((m + _BM - 1) // _BM) * _BM
    r = x if res_in is None else res_in
    if mp != m:
        x = jnp.pad(x, ((0, mp - m), (0, 0)))
        if residual:
            r = jnp.pad(r, ((0, mp - m), (0, 0)))
    grid = (mp // _BM,)
    in_specs = [
        pl.BlockSpec((_BM, k), lambda i: (i, 0)),
        pl.BlockSpec((k, n), lambda i: (0, 0)),
        pl.BlockSpec((1, n), lambda i: (0, 0)),
        pl.BlockSpec((_BM, n), lambda i: (i, 0)),
    ]
    out = pl.pallas_call(
        functools.partial(_linear_body, act=act, residual=residual,
                          add_then_act=add_then_act),
        grid=grid,
        in_specs=in_specs,
        out_specs=pl.BlockSpec((_BM, n), lambda i: (i, 0)),
        out_shape=jax.ShapeDtypeStruct((mp, n), jnp.float32),
    )(x, w, b.reshape(1, n), r if residual else jnp.zeros((mp, n), jnp.float32))
    return out[:m] if mp != m else out


def _leaky(x):
    return jnp.where(x >= 0, x, ALPHA * x)


def _relu(x):
    return jnp.maximum(x, 0.0)


_BQ = 256


def _knn_body(q_ref, rt_ref, o_ref, *, k, r_n):
    q = q_ref[...]
    rt = rt_ref[...]
    qq = jnp.sum(q * q, axis=1, keepdims=True)
    rsq = jnp.sum(rt * rt, axis=0, keepdims=True)
    qr = jnp.dot(q.astype(jnp.bfloat16), rt.astype(jnp.bfloat16),
                 preferred_element_type=jnp.float32)
    d = qq - 2.0 * qr + rsq
    iota = jax.lax.broadcasted_iota(jnp.int32, d.shape, 1)
    big = jnp.float32(jnp.inf)
    for j in range(k):
        dmin = jnp.min(d, axis=1, keepdims=True)
        sel = jnp.where(d == dmin, iota, r_n)
        ij = jnp.min(sel, axis=1)
        o_ref[:, j] = ij
        d = jnp.where(iota == ij[:, None], big, d)


def _knn_idx(q, r, k):
    qn, rn = q.shape[0], r.shape[0]
    qp = ((qn + _BQ - 1) // _BQ) * _BQ
    if qp != qn:
        q = jnp.pad(q, ((0, qp - qn), (0, 0)))
    rt = r.T
    idx = pl.pallas_call(
        functools.partial(_knn_body, k=k, r_n=rn),
        grid=(qp // _BQ,),
        in_specs=[
            pl.BlockSpec((_BQ, 3), lambda i: (i, 0)),
            pl.BlockSpec((3, rn), lambda i: (0, 0)),
        ],
        out_specs=pl.BlockSpec((_BQ, k), lambda i: (i, 0)),
        out_shape=jax.ShapeDtypeStruct((qp, k), jnp.int32),
    )(q, rt)
    return idx[:qn] if qp != qn else idx


def _gapl_edges(pos_s, x_s, ei, pos_d, x_d, p, name):
    dst, src = ei[0], ei[1]
    c = x_s.shape[1]
    w = p[name + "_W"]
    xw = _linear(x_s, w[:c], jnp.zeros((w.shape[1],), jnp.float32))
    pw = _linear(pos_s[src] - pos_d[dst], w[c:], p[name + "_b"])
    m = _leaky(xw[src] + pw)
    n_d = x_d.shape[0]
    agg = jax.ops.segment_sum(m, dst, num_segments=n_d)
    deg = jax.ops.segment_sum(jnp.ones((dst.shape[0],), jnp.float32), dst, num_segments=n_d)
    agg = agg / jnp.clip(deg, 1.0)[:, None]
    out = BETA * x_d + agg
    for blk in ("_r0", "_r1"):
        out = _linear(out, p[name + blk + "_W"], p[name + blk + "_b"],
                      act="relu", residual=True)
    return out


def _gapl_knn(pos_s, x_s, idx, pos_d, x_d, p, name, mask=None):
    nk, k = idx.shape
    c = x_s.shape[1]
    w = p[name + "_W"]
    xw = _linear(x_s, w[:c], jnp.zeros((w.shape[1],), jnp.float32))
    pd = (pos_s[idx] - pos_d[:, None, :]).reshape(nk * k, 3)
    pw = _linear(pd, w[c:], p[name + "_b"])
    m = _leaky(xw[idx].reshape(nk * k, -1) + pw).reshape(nk, k, -1)
    if mask is None:
        agg = m.sum(axis=1) / float(k)
    else:
        m = m * mask[:, :, None]
        deg = mask.sum(axis=1)
        agg = m.sum(axis=1) / jnp.clip(deg, 1.0)[:, None]
    out = BETA * x_d + agg
    for blk in ("_r0", "_r1"):
        out = _linear(out, p[name + blk + "_W"], p[name + blk + "_b"],
                      act="relu", residual=True)
    return out


def _mesh_enc(tok, p, pre):
    n = tok.shape[0]
    h = _linear(tok.reshape(n * 4, 3), p[pre + "_tok_W"], p[pre + "_tok_b"], act="relu")
    h = h.reshape(n, 4, -1) + p[pre + "_pos"][None]
    h = h.mean(axis=1)
    return _linear(h, p[pre + "_out_W"], p[pre + "_out_b"], act="relu")


def _knn_interpolate(x, pos_c, pos_f, k=3):
    idx = _knn_idx(pos_f, pos_c, k)
    d2 = jnp.sum((pos_f[:, None, :] - pos_c[idx]) ** 2, -1)
    w = 1.0 / (d2 + 1e-8)
    return jnp.sum(w[..., None] * x[idx], axis=1) / jnp.sum(w, axis=1, keepdims=True)


def kernel(pos, rgb, normals, params, edge_index, batch, ptr):
    p = params
    x_rgb = _mesh_enc(rgb.reshape(-1, 4, 3), p, "ce")
    x_nrm = _mesh_enc(normals.reshape(-1, 4, 3), p, "ne")
    x = jnp.concatenate([x_rgb, x_nrm, pos], -1)
    x = _linear(x, p["emb_W"], p["emb_b"], act="relu")
    x = _gapl_edges(pos, x, edge_index, pos, x, p, "g0")
    for blk in ("rm0", "rm1"):
        x = _linear(x, p[blk + "_W"], p[blk + "_b"], act="relu", residual=True)
    pos_down, x_down = [pos], [x]
    n = pos.shape[0]
    for i in range(NUM_CONVS):
        n_keep = int(n * POOL[i])
        perm = jax.random.permutation(jax.random.key(100 + i), n)[:n_keep]
        pos_pool = pos[perm]
        idx_h = _knn_idx(pos_pool, pos, K_NBR)
        x_hier = _gapl_knn(pos, x, idx_h, pos_pool, x[perm], p, f"hi{i}")
        idx_l = _knn_idx(pos_pool, pos_pool, K_NBR)
        sim = jnp.sum(x_hier[:, None, :] * x_hier[idx_l], -1)
        n_e = n_keep * K_NBR
        _, keep = jax.lax.top_k(sim.reshape(-1), n_e // 2)
        mask = jnp.zeros((n_e,), jnp.float32).at[keep].set(1.0).reshape(n_keep, K_NBR)
        x_local = _gapl_knn(pos_pool, x_hier, idx_l, pos_pool, x_hier, p, f"lo{i}",
                            mask=mask)
        x = jnp.concatenate([x_local, x_hier], -1)
        for blk in (f"rc{i}_0", f"rc{i}_1"):
            x = _linear(x, p[blk + "_W"], p[blk + "_b"], act="relu", residual=True)
        pos, n = pos_pool, n_keep
        pos_down.append(pos)
        x_down.append(x)
    xs = x_down[::-1]
    poss = pos_down[::-1]
    x_i = xs[0]
    for i in range(NUM_CONVS):
        up = _knn_interpolate(x_i, poss[i], poss[i + 1], 3)
        x_i = jnp.concatenate([xs[i + 1], up], -1)
        x_i = _linear(x_i, p[f"up{i}_W"], p[f"up{i}_b"], act="relu")
        x_i = _linear(x_i, p[f"up{i}_r_W"], p[f"up{i}_r_b"], act="relu", residual=True)
    y = _linear(x_i, p["m0_W"], p["m0_b"], act="relu")
    y = _linear(y, p["m1_W"], p["m1_b"], act="relu")
    return _linear(y, p["m2_W"], p["m2_b"])
